# Initial kernel scaffold; baseline (speedup 1.0000x reference)
import jax
import jax.numpy as jnp
from jax.experimental import pallas as pl


def kernel(x, lengths, tfidf, table, W1, b1, W2, b2):
    raise NotImplementedError("write your pallas kernel here")


# speedup vs baseline: 7.7961x; 7.7961x over previous
import functools

import jax
import jax.numpy as jnp
from jax import lax
from jax.experimental import pallas as pl
from jax.experimental.pallas import tpu as pltpu
from jax.experimental.pallas import tpu_sc as plsc

B = 4096
SEQ = 50
D = 64
HID = 50
OUT = 10
LANES = 16
NC = 2
NS = 16
NW = NC * NS
BPW = B // NW
CB = 8
NCHUNK = BPW // CB
SEQP = 64
DCH = D // LANES


def _bcast_lane(vec, lane):
    idx = jnp.full((LANES, 1), lane, jnp.int32)
    dnums = lax.GatherDimensionNumbers(
        offset_dims=(), collapsed_slice_dims=(0,), start_index_map=(0,))
    return lax.gather(vec, idx, dnums, (1,),
                      mode=lax.GatherScatterMode.PROMISE_IN_BOUNDS)


def _pool_body(x_hbm, tf_hbm, table_hbm, out_hbm, idx_all, emb_v, w_v,
               out_v, sem):
    wid = lax.axis_index("s") * NC + lax.axis_index("c")
    base = wid * BPW

    pltpu.sync_copy(x_hbm.at[pl.ds(base, BPW)], idx_all)

    def chunk_body(ci, carry):
        row0 = ci * CB
        copies = []
        for rr in range(CB):
            copies.append(pltpu.async_copy(
                table_hbm.at[idx_all.at[row0 + rr]],
                emb_v.at[pl.ds(rr * SEQ, SEQ)],
                sem,
            ))
        pltpu.sync_copy(tf_hbm.at[pl.ds(base + row0, CB)], w_v)
        for cp in copies:
            cp.wait()

        def row_body(rr, carry2):
            wch = [w_v[rr, pl.ds(c * LANES, LANES)] for c in range(DCH)]
            accs = [jnp.zeros((LANES,), jnp.float32) for _ in range(DCH)]
            for l in range(SEQ):
                wl = _bcast_lane(wch[l // LANES], l % LANES)
                t = rr * SEQ + l
                for c in range(DCH):
                    accs[c] = accs[c] + wl * emb_v[t, pl.ds(c * LANES, LANES)]
            for c in range(DCH):
                out_v[row0 + rr, pl.ds(c * LANES, LANES)] = accs[c]
            return carry2

        lax.fori_loop(0, CB, row_body, 0)
        return carry

    lax.fori_loop(0, NCHUNK, chunk_body, 0)
    pltpu.sync_copy(out_v, out_hbm.at[pl.ds(base, BPW)])


_pool = functools.partial(
    pl.kernel,
    out_type=jax.ShapeDtypeStruct((B, D), jnp.float32),
    mesh=plsc.VectorSubcoreMesh(
        core_axis_name="c", subcore_axis_name="s",
        num_cores=NC, num_subcores=NS),
    scratch_types=[
        pltpu.VMEM((BPW, SEQ), jnp.int32),
        pltpu.VMEM((CB * SEQ, D), jnp.float32),
        pltpu.VMEM((CB, SEQP), jnp.float32),
        pltpu.VMEM((BPW, D), jnp.float32),
        pltpu.SemaphoreType.DMA,
    ],
    compiler_params=pltpu.CompilerParams(use_tc_tiling_on_sc=False),
)(_pool_body)


def _mlp_body(racc_ref, tf_ref, len_ref, w1_ref, b1_ref, w2_ref, b2_ref,
              out_ref):
    denom = jnp.sum(tf_ref[...], axis=1, keepdims=True)
    scale = 1.0 / (denom * len_ref[...])
    rep = racc_ref[...] * scale
    h = lax.dot_general(rep, w1_ref[...], (((1,), (1,)), ((), ())),
                        preferred_element_type=jnp.float32)
    h = jnp.maximum(h + b1_ref[...], 0.0)
    logits = lax.dot_general(h, w2_ref[...], (((1,), (1,)), ((), ())),
                             preferred_element_type=jnp.float32)
    out_ref[...] = logits + b2_ref[...]


def kernel(x, lengths, tfidf, table, W1, b1, W2, b2):
    xi = x.astype(jnp.int32)
    tf_pad = jnp.pad(tfidf, ((0, 0), (0, SEQP - SEQ)))
    racc = _pool(xi, tf_pad, table)
    len_f = lengths.astype(jnp.float32).reshape(B, 1)
    return pl.pallas_call(
        _mlp_body,
        out_shape=jax.ShapeDtypeStruct((B, OUT), jnp.float32),
    )(racc, tfidf, len_f, W1, b1.reshape(1, HID), W2, b2.reshape(1, OUT))
